# Initial kernel scaffold; baseline (speedup 1.0000x reference)
import jax
import jax.numpy as jnp
from jax.experimental import pallas as pl


def kernel(x, edge_index, batch, W_in, b_in, Wc1, bc1, g1, be1, Wc2, bc2, g2, be2, Wc3, bc3, g3, be3, W_out, b_out):
    raise NotImplementedError("write your pallas kernel here")


# speedup vs baseline: 9.3449x; 9.3449x over previous
import functools

import jax
import jax.numpy as jnp
from jax import lax
from jax.experimental import pallas as pl
from jax.experimental.pallas import tpu as pltpu
from jax.experimental.pallas import tpu_sc as plsc

N = 10000
E = 320000
D = 128
G = 256
EPS = 1e-5

NP = 10240
RB = 512
NBLK = NP // RB
NC, NS = 2, 16
NT = NC * NS
CH = 128
NCHUNK = -(-E // (NT * CH * 8)) * NT * 8
EP = NCHUNK * CH
CPT = NCHUNK // NT
RPT = NP // NS
SBN = 1.0 / (1.0 + EPS) ** 0.5
PREC = lax.Precision.HIGHEST

_MESH = plsc.VectorSubcoreMesh(
    core_axis_name="c", subcore_axis_name="s", num_cores=NC, num_subcores=NS)


def _deg_body(dst_hbm, ones_hbm, zeros_hbm, out_hbm, deg_sp, dstv, ones_v):
    c = lax.axis_index("c")
    s = lax.axis_index("s")
    t = c * NS + s
    pltpu.sync_copy(zeros_hbm, deg_sp.at[pl.ds(s * RPT, RPT)])
    pltpu.sync_copy(ones_hbm, ones_v)
    pltpu.sync_copy(dst_hbm.at[pl.ds(t * CPT, CPT)], dstv)
    plsc.subcore_barrier()

    def body(j, carry):
        pltpu.sync_copy(ones_v, deg_sp.at[dstv.at[j]], add=True)
        return carry

    lax.fori_loop(0, CPT, body, 0)
    plsc.subcore_barrier()
    pltpu.sync_copy(deg_sp.at[pl.ds(s * RPT, RPT)],
                    out_hbm.at[pl.ds(c * NP + s * RPT, RPT)])


_deg_kernel = functools.partial(
    pl.kernel,
    out_type=jax.ShapeDtypeStruct((NC * NP,), jnp.float32),
    mesh=_MESH,
    scratch_types=[
        pltpu.VMEM_SHARED((NP,), jnp.float32),
        pltpu.VMEM((CPT, CH), jnp.int32),
        pltpu.VMEM((CH,), jnp.float32),
    ],
)(_deg_body)


def _edge_body(mp_hbm, src_hbm, dst1_hbm, zeros_hbm, out_hbm,
               acc_sp, srcv, dstv, rows, semg):
    c = lax.axis_index("c")
    s = lax.axis_index("s")
    t = c * NS + s
    pltpu.sync_copy(zeros_hbm, acc_sp.at[pl.ds(s * RPT, RPT)])
    pltpu.sync_copy(src_hbm.at[pl.ds(t * CPT, CPT)], srcv)
    pltpu.sync_copy(dst1_hbm.at[pl.ds(t * CPT * CH, CH)], dstv.at[0])
    plsc.subcore_barrier()

    pltpu.async_copy(mp_hbm.at[srcv.at[0]], rows.at[0], semg)

    def body(j, carry):
        p = j % 2
        pn = (j + 1) % 2

        @pl.when(j + 1 < CPT)
        def _():
            pltpu.async_copy(mp_hbm.at[srcv.at[j + 1]], rows.at[pn], semg)
            pltpu.sync_copy(dst1_hbm.at[pl.ds((t * CPT + j + 1) * CH, CH)],
                            dstv.at[pn])

        pltpu.make_async_copy(mp_hbm.at[srcv.at[j]], rows.at[p], semg).wait()
        pltpu.sync_copy(rows.at[p], acc_sp.at[dstv.at[p]], add=True)
        return carry

    lax.fori_loop(0, CPT, body, 0)
    plsc.subcore_barrier()
    pltpu.sync_copy(acc_sp.at[pl.ds(s * RPT, RPT)],
                    out_hbm.at[c, pl.ds(s * RPT, RPT)])


_edge_kernel = functools.partial(
    pl.kernel,
    out_type=jax.ShapeDtypeStruct((NC, NP, D), jnp.float32),
    mesh=_MESH,
    scratch_types=[
        pltpu.VMEM_SHARED((NP, D), jnp.float32),
        pltpu.VMEM((CPT, CH), jnp.int32),
        pltpu.VMEM((2, CH), jnp.int32),
        pltpu.VMEM((2, CH, D), jnp.float32),
        pltpu.SemaphoreType.DMA,
    ],
)(_edge_body)


def _proj_body(x_ref, dA_ref, dB_ref, Win_ref, bin_ref, Wc_ref,
               h_ref, mp_ref, dinv_ref):
    i = pl.program_id(0)
    dinv = lax.rsqrt(dA_ref[...] + dB_ref[...] + 1.0)
    h = jnp.dot(x_ref[...], Win_ref[...], precision=PREC,
                preferred_element_type=jnp.float32) + bin_ref[...]
    mp = jnp.dot(h, Wc_ref[...], precision=PREC,
                 preferred_element_type=jnp.float32) * dinv
    rows = lax.broadcasted_iota(jnp.int32, (RB, 1), 0) + i * RB
    mp_ref[...] = jnp.where(rows < N, mp, 0.0)
    h_ref[...] = h
    dinv_ref[...] = dinv


def _layer_body(acc_ref, mp_ref, h_ref, dinv_ref, b_ref, g_ref, be_ref,
                Wn_ref, h_out, mp_out):
    i = pl.program_id(0)
    dinv = dinv_ref[...]
    agg = dinv * (acc_ref[0] + acc_ref[1] + mp_ref[...])
    t = g_ref[...] * ((agg + b_ref[...]) * SBN) + be_ref[...]
    h_new = h_ref[...] + jnp.maximum(t, 0.0)
    mp_next = jnp.dot(h_new, Wn_ref[...], precision=PREC,
                      preferred_element_type=jnp.float32) * dinv
    rows = lax.broadcasted_iota(jnp.int32, (RB, 1), 0) + i * RB
    h_out[...] = h_new
    mp_out[...] = jnp.where(rows < N, mp_next, 0.0)


def _final_body(acc_ref, mp_ref, h_ref, dinv_ref, b_ref, g_ref, be_ref,
                batch_ref, Wout_ref, bout_ref, out_ref, sums, cnt):
    i = pl.program_id(0)
    agg = dinv_ref[...] * (acc_ref[0] + acc_ref[1] + mp_ref[...])
    t = g_ref[...] * ((agg + b_ref[...]) * SBN) + be_ref[...]
    h_new = h_ref[...] + jnp.maximum(t, 0.0)

    oh = (batch_ref[...] == lax.broadcasted_iota(jnp.int32, (RB, G), 1))
    oh = oh.astype(jnp.float32)

    @pl.when(i == 0)
    def _():
        sums[...] = jnp.zeros((G, D), jnp.float32)
        cnt[...] = jnp.zeros((G, 1), jnp.float32)

    dn = (((0,), (0,)), ((), ()))
    sums[...] += lax.dot_general(oh, h_new, dn, precision=PREC,
                                 preferred_element_type=jnp.float32)
    cnt[...] += lax.dot_general(oh, jnp.ones((RB, 1), jnp.float32), dn,
                                precision=PREC,
                                preferred_element_type=jnp.float32)

    @pl.when(i == NBLK - 1)
    def _():
        emb = sums[...] / jnp.maximum(cnt[...], 1.0)
        out_ref[...] = jnp.dot(emb, Wout_ref[...], precision=PREC,
                               preferred_element_type=jnp.float32) + bout_ref[...]


def _row_spec(w):
    return pl.BlockSpec((RB, w), lambda i: (i, 0))


def _full_spec(h, w):
    return pl.BlockSpec((h, w), lambda i: (0, 0))


_proj_call = pl.pallas_call(
    _proj_body,
    grid=(NBLK,),
    in_specs=[_row_spec(D), _row_spec(1), _row_spec(1),
              _full_spec(D, D), _full_spec(1, D), _full_spec(D, D)],
    out_specs=[_row_spec(D), _row_spec(D), _row_spec(1)],
    out_shape=[jax.ShapeDtypeStruct((NP, D), jnp.float32),
               jax.ShapeDtypeStruct((NP, D), jnp.float32),
               jax.ShapeDtypeStruct((NP, 1), jnp.float32)],
)

_acc_spec = pl.BlockSpec((NC, RB, D), lambda i: (0, i, 0))

_layer_call = pl.pallas_call(
    _layer_body,
    grid=(NBLK,),
    in_specs=[_acc_spec, _row_spec(D), _row_spec(D), _row_spec(1),
              _full_spec(1, D), _full_spec(1, D), _full_spec(1, D),
              _full_spec(D, D)],
    out_specs=[_row_spec(D), _row_spec(D)],
    out_shape=[jax.ShapeDtypeStruct((NP, D), jnp.float32),
               jax.ShapeDtypeStruct((NP, D), jnp.float32)],
)

_final_call = pl.pallas_call(
    _final_body,
    grid=(NBLK,),
    in_specs=[_acc_spec, _row_spec(D), _row_spec(D), _row_spec(1),
              _full_spec(1, D), _full_spec(1, D), _full_spec(1, D),
              _row_spec(1), _full_spec(D, D), _full_spec(1, D)],
    out_specs=pl.BlockSpec((G, D), lambda i: (0, 0)),
    out_shape=jax.ShapeDtypeStruct((G, D), jnp.float32),
    scratch_shapes=[pltpu.VMEM((G, D), jnp.float32),
                    pltpu.VMEM((G, 1), jnp.float32)],
)


def kernel(x, edge_index, batch, W_in, b_in,
           Wc1, bc1, g1, be1,
           Wc2, bc2, g2, be2,
           Wc3, bc3, g3, be3,
           W_out, b_out):
    x_p = jnp.pad(x, ((0, NP - N), (0, 0)))
    src = jnp.pad(edge_index[0], (0, EP - E)).reshape(NCHUNK, CH)
    dst1 = jnp.pad(edge_index[1], (0, EP - E), constant_values=NP - 1)
    dst = dst1.reshape(NCHUNK, CH)
    batch_p = jnp.pad(batch, (0, NP - N), constant_values=G).reshape(NP, 1)
    ones_c = jnp.ones((CH,), jnp.float32)
    zeros1 = jnp.zeros((RPT,), jnp.float32)
    zeros2 = jnp.zeros((RPT, D), jnp.float32)
    bin2 = b_in.reshape(1, D)
    bout2 = b_out.reshape(1, D)

    deg = _deg_kernel(dst, ones_c, zeros1).reshape(NC, NP)
    dA = deg[0].reshape(NP, 1)
    dB = deg[1].reshape(NP, 1)

    h0, mp1, dinv = _proj_call(x_p, dA, dB, W_in, bin2, Wc1)

    acc1 = _edge_kernel(mp1, src, dst1, zeros2)
    h1, mp2 = _layer_call(acc1, mp1, h0, dinv, bc1.reshape(1, D),
                          g1.reshape(1, D), be1.reshape(1, D), Wc2)

    acc2 = _edge_kernel(mp2, src, dst1, zeros2)
    h2, mp3 = _layer_call(acc2, mp2, h1, dinv, bc2.reshape(1, D),
                          g2.reshape(1, D), be2.reshape(1, D), Wc3)

    acc3 = _edge_kernel(mp3, src, dst1, zeros2)
    out = _final_call(acc3, mp3, h2, dinv, bc3.reshape(1, D),
                      g3.reshape(1, D), be3.reshape(1, D),
                      batch_p, W_out, bout2)
    return out


# speedup vs baseline: 21.9635x; 2.3503x over previous
import functools

import jax
import jax.numpy as jnp
from jax import lax
from jax.experimental import pallas as pl
from jax.experimental.pallas import tpu as pltpu
from jax.experimental.pallas import tpu_sc as plsc

N = 10000
E = 320000
D = 128
G = 256
EPS = 1e-5

NP = 10240
RB = 512
NBLK = NP // RB
NC, NS = 2, 16
NT = NC * NS
CH = 128
NCHUNK = -(-E // (NT * CH * 8)) * NT * 8
EP = NCHUNK * CH
CPT = NCHUNK // NT
RPT = NP // NS
SBN = 1.0 / (1.0 + EPS) ** 0.5
PREC = lax.Precision.HIGHEST

_MESH = plsc.VectorSubcoreMesh(
    core_axis_name="c", subcore_axis_name="s", num_cores=NC, num_subcores=NS)


def _deg_body(dst_hbm, ones_hbm, zeros_hbm, out_hbm, deg_sp, dstv, ones_v):
    c = lax.axis_index("c")
    s = lax.axis_index("s")
    t = c * NS + s
    pltpu.sync_copy(zeros_hbm, deg_sp.at[pl.ds(s * RPT, RPT)])
    pltpu.sync_copy(ones_hbm, ones_v)
    pltpu.sync_copy(dst_hbm.at[pl.ds(t * CPT, CPT)], dstv)
    plsc.subcore_barrier()

    def body(j, carry):
        pltpu.sync_copy(ones_v, deg_sp.at[dstv.at[j]], add=True)
        return carry

    lax.fori_loop(0, CPT, body, 0)
    plsc.subcore_barrier()
    pltpu.sync_copy(deg_sp.at[pl.ds(s * RPT, RPT)],
                    out_hbm.at[pl.ds(c * NP + s * RPT, RPT)])


_deg_kernel = functools.partial(
    pl.kernel,
    out_type=jax.ShapeDtypeStruct((NC * NP,), jnp.float32),
    mesh=_MESH,
    scratch_types=[
        pltpu.VMEM_SHARED((NP,), jnp.float32),
        pltpu.VMEM((CPT, CH), jnp.int32),
        pltpu.VMEM((CH,), jnp.float32),
    ],
)(_deg_body)


def _edge_body(mp_hbm, src_hbm, dst1_hbm, zeros_hbm, out_hbm,
               acc_sp, srcv, dstv, rows, semg, sems, semi):
    c = lax.axis_index("c")
    s = lax.axis_index("s")
    t = c * NS + s
    pltpu.sync_copy(zeros_hbm, acc_sp.at[pl.ds(s * RPT, RPT)])
    pltpu.sync_copy(src_hbm.at[pl.ds(t * CPT, CPT)], srcv)
    pltpu.sync_copy(dst1_hbm.at[pl.ds(t * CPT * CH, CH)], dstv.at[0])
    plsc.subcore_barrier()

    pltpu.async_copy(mp_hbm.at[srcv.at[0]], rows.at[0], semg)

    def body(j, carry):
        p = j % 2
        pn = (j + 1) % 2
        pltpu.make_async_copy(mp_hbm.at[srcv.at[j]], rows.at[p], semg).wait()

        @pl.when(j + 1 < CPT)
        def _():
            pltpu.async_copy(mp_hbm.at[srcv.at[j + 1]], rows.at[pn], semg)
            pltpu.async_copy(
                dst1_hbm.at[pl.ds((t * CPT + j + 1) * CH, CH)],
                dstv.at[pn], semi)

        @pl.when(j >= 1)
        def _():
            pltpu.make_async_copy(dst1_hbm.at[pl.ds(t * CPT * CH, CH)],
                                  dstv.at[p], semi).wait()

        pltpu.sync_copy(rows.at[p], acc_sp.at[dstv.at[p]], add=True)
        return carry

    lax.fori_loop(0, CPT, body, 0)
    plsc.subcore_barrier()
    pltpu.sync_copy(acc_sp.at[pl.ds(s * RPT, RPT)],
                    out_hbm.at[c, pl.ds(s * RPT, RPT)])


_edge_kernel = functools.partial(
    pl.kernel,
    out_type=jax.ShapeDtypeStruct((NC, NP, D), jnp.float32),
    mesh=_MESH,
    scratch_types=[
        pltpu.VMEM_SHARED((NP, D), jnp.float32),
        pltpu.VMEM((CPT, CH), jnp.int32),
        pltpu.VMEM((2, CH), jnp.int32),
        pltpu.VMEM((2, CH, D), jnp.float32),
        pltpu.SemaphoreType.DMA,
        pltpu.SemaphoreType.DMA,
        pltpu.SemaphoreType.DMA,
    ],
)(_edge_body)


def _proj_body(x_ref, dA_ref, dB_ref, Win_ref, bin_ref, Wc_ref,
               h_ref, mp_ref, dinv_ref):
    i = pl.program_id(0)
    dinv = lax.rsqrt(dA_ref[...] + dB_ref[...] + 1.0)
    h = jnp.dot(x_ref[...], Win_ref[...], precision=PREC,
                preferred_element_type=jnp.float32) + bin_ref[...]
    mp = jnp.dot(h, Wc_ref[...], precision=PREC,
                 preferred_element_type=jnp.float32) * dinv
    rows = lax.broadcasted_iota(jnp.int32, (RB, 1), 0) + i * RB
    mp_ref[...] = jnp.where(rows < N, mp, 0.0)
    h_ref[...] = h
    dinv_ref[...] = dinv


def _layer_body(acc_ref, mp_ref, h_ref, dinv_ref, b_ref, g_ref, be_ref,
                Wn_ref, h_out, mp_out):
    i = pl.program_id(0)
    dinv = dinv_ref[...]
    agg = dinv * (acc_ref[0] + acc_ref[1] + mp_ref[...])
    t = g_ref[...] * ((agg + b_ref[...]) * SBN) + be_ref[...]
    h_new = h_ref[...] + jnp.maximum(t, 0.0)
    mp_next = jnp.dot(h_new, Wn_ref[...], precision=PREC,
                      preferred_element_type=jnp.float32) * dinv
    rows = lax.broadcasted_iota(jnp.int32, (RB, 1), 0) + i * RB
    h_out[...] = h_new
    mp_out[...] = jnp.where(rows < N, mp_next, 0.0)


def _final_body(acc_ref, mp_ref, h_ref, dinv_ref, b_ref, g_ref, be_ref,
                batch_ref, Wout_ref, bout_ref, out_ref, sums, cnt):
    i = pl.program_id(0)
    agg = dinv_ref[...] * (acc_ref[0] + acc_ref[1] + mp_ref[...])
    t = g_ref[...] * ((agg + b_ref[...]) * SBN) + be_ref[...]
    h_new = h_ref[...] + jnp.maximum(t, 0.0)

    oh = (batch_ref[...] == lax.broadcasted_iota(jnp.int32, (RB, G), 1))
    oh = oh.astype(jnp.float32)

    @pl.when(i == 0)
    def _():
        sums[...] = jnp.zeros((G, D), jnp.float32)
        cnt[...] = jnp.zeros((G, 1), jnp.float32)

    dn = (((0,), (0,)), ((), ()))
    sums[...] += lax.dot_general(oh, h_new, dn, precision=PREC,
                                 preferred_element_type=jnp.float32)
    cnt[...] += lax.dot_general(oh, jnp.ones((RB, 1), jnp.float32), dn,
                                precision=PREC,
                                preferred_element_type=jnp.float32)

    @pl.when(i == NBLK - 1)
    def _():
        emb = sums[...] / jnp.maximum(cnt[...], 1.0)
        out_ref[...] = jnp.dot(emb, Wout_ref[...], precision=PREC,
                               preferred_element_type=jnp.float32) + bout_ref[...]


def _row_spec(w):
    return pl.BlockSpec((RB, w), lambda i: (i, 0))


def _full_spec(h, w):
    return pl.BlockSpec((h, w), lambda i: (0, 0))


_proj_call = pl.pallas_call(
    _proj_body,
    grid=(NBLK,),
    in_specs=[_row_spec(D), _row_spec(1), _row_spec(1),
              _full_spec(D, D), _full_spec(1, D), _full_spec(D, D)],
    out_specs=[_row_spec(D), _row_spec(D), _row_spec(1)],
    out_shape=[jax.ShapeDtypeStruct((NP, D), jnp.float32),
               jax.ShapeDtypeStruct((NP, D), jnp.float32),
               jax.ShapeDtypeStruct((NP, 1), jnp.float32)],
)

_acc_spec = pl.BlockSpec((NC, RB, D), lambda i: (0, i, 0))

_layer_call = pl.pallas_call(
    _layer_body,
    grid=(NBLK,),
    in_specs=[_acc_spec, _row_spec(D), _row_spec(D), _row_spec(1),
              _full_spec(1, D), _full_spec(1, D), _full_spec(1, D),
              _full_spec(D, D)],
    out_specs=[_row_spec(D), _row_spec(D)],
    out_shape=[jax.ShapeDtypeStruct((NP, D), jnp.float32),
               jax.ShapeDtypeStruct((NP, D), jnp.float32)],
)

_final_call = pl.pallas_call(
    _final_body,
    grid=(NBLK,),
    in_specs=[_acc_spec, _row_spec(D), _row_spec(D), _row_spec(1),
              _full_spec(1, D), _full_spec(1, D), _full_spec(1, D),
              _row_spec(1), _full_spec(D, D), _full_spec(1, D)],
    out_specs=pl.BlockSpec((G, D), lambda i: (0, 0)),
    out_shape=jax.ShapeDtypeStruct((G, D), jnp.float32),
    scratch_shapes=[pltpu.VMEM((G, D), jnp.float32),
                    pltpu.VMEM((G, 1), jnp.float32)],
)


def kernel(x, edge_index, batch, W_in, b_in,
           Wc1, bc1, g1, be1,
           Wc2, bc2, g2, be2,
           Wc3, bc3, g3, be3,
           W_out, b_out):
    x_p = jnp.pad(x, ((0, NP - N), (0, 0)))
    pad_fill = N + (jnp.arange(EP - E, dtype=jnp.int32) % (NP - N))
    src = jnp.concatenate([edge_index[0], pad_fill]).reshape(NCHUNK, CH)
    dst1 = jnp.concatenate([edge_index[1], pad_fill])
    dst = dst1.reshape(NCHUNK, CH)
    batch_p = jnp.pad(batch, (0, NP - N), constant_values=G).reshape(NP, 1)
    ones_c = jnp.ones((CH,), jnp.float32)
    zeros1 = jnp.zeros((RPT,), jnp.float32)
    zeros2 = jnp.zeros((RPT, D), jnp.float32)
    bin2 = b_in.reshape(1, D)
    bout2 = b_out.reshape(1, D)

    deg = _deg_kernel(dst, ones_c, zeros1).reshape(NC, NP)
    dA = deg[0].reshape(NP, 1)
    dB = deg[1].reshape(NP, 1)

    h0, mp1, dinv = _proj_call(x_p, dA, dB, W_in, bin2, Wc1)

    acc1 = _edge_kernel(mp1, src, dst1, zeros2)
    h1, mp2 = _layer_call(acc1, mp1, h0, dinv, bc1.reshape(1, D),
                          g1.reshape(1, D), be1.reshape(1, D), Wc2)

    acc2 = _edge_kernel(mp2, src, dst1, zeros2)
    h2, mp3 = _layer_call(acc2, mp2, h1, dinv, bc2.reshape(1, D),
                          g2.reshape(1, D), be2.reshape(1, D), Wc3)

    acc3 = _edge_kernel(mp3, src, dst1, zeros2)
    out = _final_call(acc3, mp3, h2, dinv, bc3.reshape(1, D),
                      g3.reshape(1, D), be3.reshape(1, D),
                      batch_p, W_out, bout2)
    return out


# speedup vs baseline: 26.5750x; 1.2100x over previous
import functools

import jax
import jax.numpy as jnp
from jax import lax
from jax.experimental import pallas as pl
from jax.experimental.pallas import tpu as pltpu
from jax.experimental.pallas import tpu_sc as plsc

N = 10000
E = 320000
D = 128
G = 256
EPS = 1e-5

NP = 10240
RB = 5120
NBLK = NP // RB
NC, NS = 2, 16
NT = NC * NS
CH = 128
NCHUNK = -(-E // (NT * CH * 8)) * NT * 8
EP = NCHUNK * CH
CPT = NCHUNK // NT
RPT = NP // NS
SBN = 1.0 / (1.0 + EPS) ** 0.5
PREC = lax.Precision.HIGHEST

_MESH = plsc.VectorSubcoreMesh(
    core_axis_name="c", subcore_axis_name="s", num_cores=NC, num_subcores=NS)


def _deg_body(dst_hbm, ones_hbm, zeros_hbm, out_hbm, deg_sp, dstv, ones_v):
    c = lax.axis_index("c")
    s = lax.axis_index("s")
    t = c * NS + s
    pltpu.sync_copy(zeros_hbm, deg_sp.at[pl.ds(s * RPT, RPT)])
    pltpu.sync_copy(ones_hbm, ones_v)
    pltpu.sync_copy(dst_hbm.at[pl.ds(t * CPT, CPT)], dstv)
    plsc.subcore_barrier()

    def body(j, carry):
        pltpu.sync_copy(ones_v, deg_sp.at[dstv.at[j]], add=True)
        return carry

    lax.fori_loop(0, CPT, body, 0)
    plsc.subcore_barrier()
    pltpu.sync_copy(deg_sp.at[pl.ds(s * RPT, RPT)],
                    out_hbm.at[pl.ds(c * NP + s * RPT, RPT)])


_deg_kernel = functools.partial(
    pl.kernel,
    out_type=jax.ShapeDtypeStruct((NC * NP,), jnp.float32),
    mesh=_MESH,
    scratch_types=[
        pltpu.VMEM_SHARED((NP,), jnp.float32),
        pltpu.VMEM((CPT, CH), jnp.int32),
        pltpu.VMEM((CH,), jnp.float32),
    ],
)(_deg_body)


def _edge_body(mp_hbm, src_hbm, dst1_hbm, zeros_hbm, out_hbm,
               acc_sp, srcv, dstv, rows, semg, sems, semi):
    c = lax.axis_index("c")
    s = lax.axis_index("s")
    t = c * NS + s
    pltpu.sync_copy(zeros_hbm, acc_sp.at[pl.ds(s * RPT, RPT)])
    pltpu.sync_copy(src_hbm.at[pl.ds(t * CPT, CPT)], srcv)
    pltpu.sync_copy(dst1_hbm.at[pl.ds(t * CPT * CH, CH)], dstv.at[0])
    plsc.subcore_barrier()

    pltpu.async_copy(mp_hbm.at[srcv.at[0]], rows.at[0], semg)

    def body(j, carry):
        p = j % 2
        pn = (j + 1) % 2

        @pl.when(j >= 1)
        def _():
            pltpu.make_async_copy(mp_hbm.at[srcv.at[0]], rows.at[pn],
                                  sems).wait()

        @pl.when(j + 1 < CPT)
        def _():
            pltpu.async_copy(mp_hbm.at[srcv.at[j + 1]], rows.at[pn], semg)
            pltpu.async_copy(
                dst1_hbm.at[pl.ds((t * CPT + j + 1) * CH, CH)],
                dstv.at[pn], semi)

        pltpu.make_async_copy(mp_hbm.at[srcv.at[j]], rows.at[p], semg).wait()

        @pl.when(j >= 1)
        def _():
            pltpu.make_async_copy(dst1_hbm.at[pl.ds(t * CPT * CH, CH)],
                                  dstv.at[p], semi).wait()

        pltpu.async_copy(rows.at[p], acc_sp.at[dstv.at[p]], sems, add=True)
        return carry

    lax.fori_loop(0, CPT, body, 0)
    pltpu.make_async_copy(mp_hbm.at[srcv.at[0]], rows.at[(CPT - 1) % 2],
                          sems).wait()
    plsc.subcore_barrier()
    pltpu.sync_copy(acc_sp.at[pl.ds(s * RPT, RPT)],
                    out_hbm.at[c, pl.ds(s * RPT, RPT)])


_edge_kernel = functools.partial(
    pl.kernel,
    out_type=jax.ShapeDtypeStruct((NC, NP, D), jnp.float32),
    mesh=_MESH,
    scratch_types=[
        pltpu.VMEM_SHARED((NP, D), jnp.float32),
        pltpu.VMEM((CPT, CH), jnp.int32),
        pltpu.VMEM((2, CH), jnp.int32),
        pltpu.VMEM((2, CH, D), jnp.float32),
        pltpu.SemaphoreType.DMA,
        pltpu.SemaphoreType.DMA,
        pltpu.SemaphoreType.DMA,
    ],
)(_edge_body)


def _proj_body(x_ref, dA_ref, dB_ref, Win_ref, bin_ref, Wc_ref,
               h_ref, mp_ref, dinv_ref):
    i = pl.program_id(0)
    dinv = lax.rsqrt(dA_ref[...] + dB_ref[...] + 1.0)
    h = jnp.dot(x_ref[...], Win_ref[...], precision=PREC,
                preferred_element_type=jnp.float32) + bin_ref[...]
    mp = jnp.dot(h, Wc_ref[...], precision=PREC,
                 preferred_element_type=jnp.float32) * dinv
    rows = lax.broadcasted_iota(jnp.int32, (RB, 1), 0) + i * RB
    mp_ref[...] = jnp.where(rows < N, mp, 0.0)
    h_ref[...] = h
    dinv_ref[...] = dinv


def _layer_body(acc_ref, mp_ref, h_ref, dinv_ref, b_ref, g_ref, be_ref,
                Wn_ref, h_out, mp_out):
    i = pl.program_id(0)
    dinv = dinv_ref[...]
    agg = dinv * (acc_ref[0] + acc_ref[1] + mp_ref[...])
    t = g_ref[...] * ((agg + b_ref[...]) * SBN) + be_ref[...]
    h_new = h_ref[...] + jnp.maximum(t, 0.0)
    mp_next = jnp.dot(h_new, Wn_ref[...], precision=PREC,
                      preferred_element_type=jnp.float32) * dinv
    rows = lax.broadcasted_iota(jnp.int32, (RB, 1), 0) + i * RB
    h_out[...] = h_new
    mp_out[...] = jnp.where(rows < N, mp_next, 0.0)


def _final_body(acc_ref, mp_ref, h_ref, dinv_ref, b_ref, g_ref, be_ref,
                batch_ref, Wout_ref, bout_ref, out_ref, sums, cnt):
    i = pl.program_id(0)
    agg = dinv_ref[...] * (acc_ref[0] + acc_ref[1] + mp_ref[...])
    t = g_ref[...] * ((agg + b_ref[...]) * SBN) + be_ref[...]
    h_new = h_ref[...] + jnp.maximum(t, 0.0)

    oh = (batch_ref[...] == lax.broadcasted_iota(jnp.int32, (RB, G), 1))
    oh = oh.astype(jnp.float32)

    @pl.when(i == 0)
    def _():
        sums[...] = jnp.zeros((G, D), jnp.float32)
        cnt[...] = jnp.zeros((G, 1), jnp.float32)

    dn = (((0,), (0,)), ((), ()))
    sums[...] += lax.dot_general(oh, h_new, dn, precision=PREC,
                                 preferred_element_type=jnp.float32)
    cnt[...] += lax.dot_general(oh, jnp.ones((RB, 1), jnp.float32), dn,
                                precision=PREC,
                                preferred_element_type=jnp.float32)

    @pl.when(i == NBLK - 1)
    def _():
        emb = sums[...] / jnp.maximum(cnt[...], 1.0)
        out_ref[...] = jnp.dot(emb, Wout_ref[...], precision=PREC,
                               preferred_element_type=jnp.float32) + bout_ref[...]


def _row_spec(w):
    return pl.BlockSpec((RB, w), lambda i: (i, 0))


def _full_spec(h, w):
    return pl.BlockSpec((h, w), lambda i: (0, 0))


_proj_call = pl.pallas_call(
    _proj_body,
    grid=(NBLK,),
    in_specs=[_row_spec(D), _row_spec(1), _row_spec(1),
              _full_spec(D, D), _full_spec(1, D), _full_spec(D, D)],
    out_specs=[_row_spec(D), _row_spec(D), _row_spec(1)],
    out_shape=[jax.ShapeDtypeStruct((NP, D), jnp.float32),
               jax.ShapeDtypeStruct((NP, D), jnp.float32),
               jax.ShapeDtypeStruct((NP, 1), jnp.float32)],
)

_acc_spec = pl.BlockSpec((NC, RB, D), lambda i: (0, i, 0))

_layer_call = pl.pallas_call(
    _layer_body,
    grid=(NBLK,),
    in_specs=[_acc_spec, _row_spec(D), _row_spec(D), _row_spec(1),
              _full_spec(1, D), _full_spec(1, D), _full_spec(1, D),
              _full_spec(D, D)],
    out_specs=[_row_spec(D), _row_spec(D)],
    out_shape=[jax.ShapeDtypeStruct((NP, D), jnp.float32),
               jax.ShapeDtypeStruct((NP, D), jnp.float32)],
)

_final_call = pl.pallas_call(
    _final_body,
    grid=(NBLK,),
    in_specs=[_acc_spec, _row_spec(D), _row_spec(D), _row_spec(1),
              _full_spec(1, D), _full_spec(1, D), _full_spec(1, D),
              _row_spec(1), _full_spec(D, D), _full_spec(1, D)],
    out_specs=pl.BlockSpec((G, D), lambda i: (0, 0)),
    out_shape=jax.ShapeDtypeStruct((G, D), jnp.float32),
    scratch_shapes=[pltpu.VMEM((G, D), jnp.float32),
                    pltpu.VMEM((G, 1), jnp.float32)],
)


def kernel(x, edge_index, batch, W_in, b_in,
           Wc1, bc1, g1, be1,
           Wc2, bc2, g2, be2,
           Wc3, bc3, g3, be3,
           W_out, b_out):
    x_p = jnp.pad(x, ((0, NP - N), (0, 0)))
    pad_fill = N + (jnp.arange(EP - E, dtype=jnp.int32) % (NP - N))
    src = jnp.concatenate([edge_index[0], pad_fill]).reshape(NCHUNK, CH)
    dst1 = jnp.concatenate([edge_index[1], pad_fill])
    dst = dst1.reshape(NCHUNK, CH)
    batch_p = jnp.pad(batch, (0, NP - N), constant_values=G).reshape(NP, 1)
    ones_c = jnp.ones((CH,), jnp.float32)
    zeros1 = jnp.zeros((RPT,), jnp.float32)
    zeros2 = jnp.zeros((RPT, D), jnp.float32)
    bin2 = b_in.reshape(1, D)
    bout2 = b_out.reshape(1, D)

    deg = _deg_kernel(dst, ones_c, zeros1).reshape(NC, NP)
    dA = deg[0].reshape(NP, 1)
    dB = deg[1].reshape(NP, 1)

    h0, mp1, dinv = _proj_call(x_p, dA, dB, W_in, bin2, Wc1)

    acc1 = _edge_kernel(mp1, src, dst1, zeros2)
    h1, mp2 = _layer_call(acc1, mp1, h0, dinv, bc1.reshape(1, D),
                          g1.reshape(1, D), be1.reshape(1, D), Wc2)

    acc2 = _edge_kernel(mp2, src, dst1, zeros2)
    h2, mp3 = _layer_call(acc2, mp2, h1, dinv, bc2.reshape(1, D),
                          g2.reshape(1, D), be2.reshape(1, D), Wc3)

    acc3 = _edge_kernel(mp3, src, dst1, zeros2)
    out = _final_call(acc3, mp3, h2, dinv, bc3.reshape(1, D),
                      g3.reshape(1, D), be3.reshape(1, D),
                      batch_p, W_out, bout2)
    return out


# speedup vs baseline: 26.7304x; 1.0058x over previous
import functools

import jax
import jax.numpy as jnp
from jax import lax
from jax.experimental import pallas as pl
from jax.experimental.pallas import tpu as pltpu
from jax.experimental.pallas import tpu_sc as plsc

N = 10000
E = 320000
D = 128
G = 256
EPS = 1e-5

NP = 10240
RB = 2048
NBLK = NP // RB
NC, NS = 2, 16
NT = NC * NS
CH = 128
NCHUNK = -(-E // (NT * CH * 8)) * NT * 8
EP = NCHUNK * CH
CPT = NCHUNK // NT
RPT = NP // NS
SBN = 1.0 / (1.0 + EPS) ** 0.5
PREC = lax.Precision.HIGHEST

_MESH = plsc.VectorSubcoreMesh(
    core_axis_name="c", subcore_axis_name="s", num_cores=NC, num_subcores=NS)


def _deg_body(dst_hbm, ones_hbm, zeros_hbm, out_hbm, deg_sp, dstv, ones_v):
    c = lax.axis_index("c")
    s = lax.axis_index("s")
    t = c * NS + s
    pltpu.sync_copy(zeros_hbm, deg_sp.at[pl.ds(s * RPT, RPT)])
    pltpu.sync_copy(ones_hbm, ones_v)
    pltpu.sync_copy(dst_hbm.at[pl.ds(t * CPT, CPT)], dstv)
    plsc.subcore_barrier()

    def body(j, carry):
        pltpu.sync_copy(ones_v, deg_sp.at[dstv.at[j]], add=True)
        return carry

    lax.fori_loop(0, CPT, body, 0)
    plsc.subcore_barrier()
    pltpu.sync_copy(deg_sp.at[pl.ds(s * RPT, RPT)],
                    out_hbm.at[pl.ds(c * NP + s * RPT, RPT)])


_deg_kernel = functools.partial(
    pl.kernel,
    out_type=jax.ShapeDtypeStruct((NC * NP,), jnp.float32),
    mesh=_MESH,
    scratch_types=[
        pltpu.VMEM_SHARED((NP,), jnp.float32),
        pltpu.VMEM((CPT, CH), jnp.int32),
        pltpu.VMEM((CH,), jnp.float32),
    ],
)(_deg_body)


def _edge_body(mp_hbm, src_hbm, dst1_hbm, zeros_hbm, out_hbm,
               acc_sp, srcv, dstv, rows, semg, sems, semi):
    c = lax.axis_index("c")
    s = lax.axis_index("s")
    t = c * NS + s
    pltpu.sync_copy(zeros_hbm, rows.at[0])
    for k in range(RPT // CH):
        pltpu.sync_copy(rows.at[0], acc_sp.at[pl.ds(s * RPT + k * CH, CH)])
    pltpu.sync_copy(src_hbm.at[pl.ds(t * CPT, CPT)], srcv)
    pltpu.sync_copy(dst1_hbm.at[pl.ds(t * CPT * CH, CH)], dstv.at[0])
    plsc.subcore_barrier()

    pltpu.async_copy(mp_hbm.at[srcv.at[0]], rows.at[0], semg)

    def body(j, carry):
        p = j % 2
        pn = (j + 1) % 2

        @pl.when(j >= 1)
        def _():
            pltpu.make_async_copy(mp_hbm.at[srcv.at[0]], rows.at[pn],
                                  sems).wait()

        @pl.when(j + 1 < CPT)
        def _():
            pltpu.async_copy(mp_hbm.at[srcv.at[j + 1]], rows.at[pn], semg)
            pltpu.async_copy(
                dst1_hbm.at[pl.ds((t * CPT + j + 1) * CH, CH)],
                dstv.at[pn], semi)

        pltpu.make_async_copy(mp_hbm.at[srcv.at[j]], rows.at[p], semg).wait()

        @pl.when(j >= 1)
        def _():
            pltpu.make_async_copy(dst1_hbm.at[pl.ds(t * CPT * CH, CH)],
                                  dstv.at[p], semi).wait()

        pltpu.async_copy(rows.at[p], acc_sp.at[dstv.at[p]], sems, add=True)
        return carry

    lax.fori_loop(0, CPT, body, 0)
    pltpu.make_async_copy(mp_hbm.at[srcv.at[0]], rows.at[(CPT - 1) % 2],
                          sems).wait()
    plsc.subcore_barrier()
    pltpu.sync_copy(acc_sp.at[pl.ds(s * RPT, RPT)],
                    out_hbm.at[c, pl.ds(s * RPT, RPT)])


_edge_kernel = functools.partial(
    pl.kernel,
    out_type=jax.ShapeDtypeStruct((NC, NP, D), jnp.float32),
    mesh=_MESH,
    scratch_types=[
        pltpu.VMEM_SHARED((NP, D), jnp.float32),
        pltpu.VMEM((CPT, CH), jnp.int32),
        pltpu.VMEM((2, CH), jnp.int32),
        pltpu.VMEM((2, CH, D), jnp.float32),
        pltpu.SemaphoreType.DMA,
        pltpu.SemaphoreType.DMA,
        pltpu.SemaphoreType.DMA,
    ],
)(_edge_body)


def _proj_body(x_ref, dA_ref, dB_ref, Win_ref, bin_ref, Wc_ref,
               h_ref, mp_ref, dinv_ref):
    i = pl.program_id(0)
    dinv = lax.rsqrt(dA_ref[...] + dB_ref[...] + 1.0)
    h = jnp.dot(x_ref[...], Win_ref[...], precision=PREC,
                preferred_element_type=jnp.float32) + bin_ref[...]
    mp = jnp.dot(h, Wc_ref[...], precision=PREC,
                 preferred_element_type=jnp.float32) * dinv
    rows = lax.broadcasted_iota(jnp.int32, (RB, 1), 0) + i * RB
    mp_ref[...] = jnp.where(rows < N, mp, 0.0)
    h_ref[...] = h
    dinv_ref[...] = dinv


def _layer_body(acc_ref, mp_ref, h_ref, dinv_ref, b_ref, g_ref, be_ref,
                Wn_ref, h_out, mp_out):
    i = pl.program_id(0)
    dinv = dinv_ref[...]
    agg = dinv * (acc_ref[0] + acc_ref[1] + mp_ref[...])
    t = g_ref[...] * ((agg + b_ref[...]) * SBN) + be_ref[...]
    h_new = h_ref[...] + jnp.maximum(t, 0.0)
    mp_next = jnp.dot(h_new, Wn_ref[...], precision=PREC,
                      preferred_element_type=jnp.float32) * dinv
    rows = lax.broadcasted_iota(jnp.int32, (RB, 1), 0) + i * RB
    h_out[...] = h_new
    mp_out[...] = jnp.where(rows < N, mp_next, 0.0)


def _final_body(acc_ref, mp_ref, h_ref, dinv_ref, b_ref, g_ref, be_ref,
                batch_ref, Wout_ref, bout_ref, out_ref, sums, cnt):
    i = pl.program_id(0)
    agg = dinv_ref[...] * (acc_ref[0] + acc_ref[1] + mp_ref[...])
    t = g_ref[...] * ((agg + b_ref[...]) * SBN) + be_ref[...]
    h_new = h_ref[...] + jnp.maximum(t, 0.0)

    oh = (batch_ref[...] == lax.broadcasted_iota(jnp.int32, (RB, G), 1))
    oh = oh.astype(jnp.float32)

    @pl.when(i == 0)
    def _():
        sums[...] = jnp.zeros((G, D), jnp.float32)
        cnt[...] = jnp.zeros((G, 1), jnp.float32)

    dn = (((0,), (0,)), ((), ()))
    sums[...] += lax.dot_general(oh, h_new, dn, precision=PREC,
                                 preferred_element_type=jnp.float32)
    cnt[...] += lax.dot_general(oh, jnp.ones((RB, 1), jnp.float32), dn,
                                precision=PREC,
                                preferred_element_type=jnp.float32)

    @pl.when(i == NBLK - 1)
    def _():
        emb = sums[...] / jnp.maximum(cnt[...], 1.0)
        out_ref[...] = jnp.dot(emb, Wout_ref[...], precision=PREC,
                               preferred_element_type=jnp.float32) + bout_ref[...]


def _row_spec(w):
    return pl.BlockSpec((RB, w), lambda i: (i, 0))


def _full_spec(h, w):
    return pl.BlockSpec((h, w), lambda i: (0, 0))


_proj_call = pl.pallas_call(
    _proj_body,
    grid=(NBLK,),
    in_specs=[_row_spec(D), _row_spec(1), _row_spec(1),
              _full_spec(D, D), _full_spec(1, D), _full_spec(D, D)],
    out_specs=[_row_spec(D), _row_spec(D), _row_spec(1)],
    out_shape=[jax.ShapeDtypeStruct((NP, D), jnp.float32),
               jax.ShapeDtypeStruct((NP, D), jnp.float32),
               jax.ShapeDtypeStruct((NP, 1), jnp.float32)],
)

_acc_spec = pl.BlockSpec((NC, RB, D), lambda i: (0, i, 0))

_layer_call = pl.pallas_call(
    _layer_body,
    grid=(NBLK,),
    in_specs=[_acc_spec, _row_spec(D), _row_spec(D), _row_spec(1),
              _full_spec(1, D), _full_spec(1, D), _full_spec(1, D),
              _full_spec(D, D)],
    out_specs=[_row_spec(D), _row_spec(D)],
    out_shape=[jax.ShapeDtypeStruct((NP, D), jnp.float32),
               jax.ShapeDtypeStruct((NP, D), jnp.float32)],
)

_final_call = pl.pallas_call(
    _final_body,
    grid=(NBLK,),
    in_specs=[_acc_spec, _row_spec(D), _row_spec(D), _row_spec(1),
              _full_spec(1, D), _full_spec(1, D), _full_spec(1, D),
              _row_spec(1), _full_spec(D, D), _full_spec(1, D)],
    out_specs=pl.BlockSpec((G, D), lambda i: (0, 0)),
    out_shape=jax.ShapeDtypeStruct((G, D), jnp.float32),
    scratch_shapes=[pltpu.VMEM((G, D), jnp.float32),
                    pltpu.VMEM((G, 1), jnp.float32)],
)


def kernel(x, edge_index, batch, W_in, b_in,
           Wc1, bc1, g1, be1,
           Wc2, bc2, g2, be2,
           Wc3, bc3, g3, be3,
           W_out, b_out):
    x_p = jnp.pad(x, ((0, NP - N), (0, 0)))
    pad_fill = N + (jnp.arange(EP - E, dtype=jnp.int32) % (NP - N))
    src = jnp.concatenate([edge_index[0], pad_fill]).reshape(NCHUNK, CH)
    dst1 = jnp.concatenate([edge_index[1], pad_fill])
    dst = dst1.reshape(NCHUNK, CH)
    batch_p = jnp.pad(batch, (0, NP - N), constant_values=G).reshape(NP, 1)
    ones_c = jnp.ones((CH,), jnp.float32)
    zeros1 = jnp.zeros((RPT,), jnp.float32)
    zeros2 = jnp.zeros((CH, D), jnp.float32)
    bin2 = b_in.reshape(1, D)
    bout2 = b_out.reshape(1, D)

    deg = _deg_kernel(dst, ones_c, zeros1).reshape(NC, NP)
    dA = deg[0].reshape(NP, 1)
    dB = deg[1].reshape(NP, 1)

    h0, mp1, dinv = _proj_call(x_p, dA, dB, W_in, bin2, Wc1)

    acc1 = _edge_kernel(mp1, src, dst1, zeros2)
    h1, mp2 = _layer_call(acc1, mp1, h0, dinv, bc1.reshape(1, D),
                          g1.reshape(1, D), be1.reshape(1, D), Wc2)

    acc2 = _edge_kernel(mp2, src, dst1, zeros2)
    h2, mp3 = _layer_call(acc2, mp2, h1, dinv, bc2.reshape(1, D),
                          g2.reshape(1, D), be2.reshape(1, D), Wc3)

    acc3 = _edge_kernel(mp3, src, dst1, zeros2)
    out = _final_call(acc3, mp3, h2, dinv, bc3.reshape(1, D),
                      g3.reshape(1, D), be3.reshape(1, D),
                      batch_p, W_out, bout2)
    return out
